# initial kernel scaffold (unmeasured)
import jax
import jax.numpy as jnp
from jax import lax
from jax.experimental import pallas as pl
from jax.experimental.pallas import tpu as pltpu

N_Y = 4
F_TILE = 2048


def kernel(x, dy):
    m_shard, d = x.shape
    _, f = dy.shape
    chunk = d // N_Y

    xt = x.T.astype(jnp.bfloat16)
    dyb = dy.astype(jnp.bfloat16)

    def body(xt_ref, dy_ref, out_ref, stage_ref, recv_ref, send_sem, recv_sems):
        y = lax.axis_index("y")
        xp = lax.axis_index("x")
        zp = lax.axis_index("z")
        left = (y + N_Y - 1) % N_Y
        right = (y + 1) % N_Y

        barrier = pltpu.get_barrier_semaphore()
        for nbr in (left, right):
            pl.semaphore_signal(
                barrier, inc=1,
                device_id=(xp, nbr, zp),
                device_id_type=pl.DeviceIdType.MESH,
            )
        pl.semaphore_wait(barrier, 2)

        def compute_chunk(c, add_slot, dst_ref, f32_out):
            xrow = xt_ref[pl.ds(c * chunk, chunk), :]
            for t in range(f // F_TILE):
                sl = pl.ds(t * F_TILE, F_TILE)
                p = jnp.dot(xrow, dy_ref[:, sl],
                            preferred_element_type=jnp.float32)
                if add_slot is not None:
                    p = p + recv_ref[add_slot, :, sl].astype(jnp.float32)
                if f32_out:
                    dst_ref[:, sl] = p
                else:
                    dst_ref[:, sl] = p.astype(jnp.bfloat16)

        for h in range(N_Y - 1):
            c = (y + (N_Y - 1 - h)) % N_Y
            compute_chunk(c, None if h == 0 else h - 1, stage_ref, False)
            rdma = pltpu.make_async_remote_copy(
                src_ref=stage_ref,
                dst_ref=recv_ref.at[h],
                send_sem=send_sem,
                recv_sem=recv_sems.at[h],
                device_id=(xp, right, zp),
                device_id_type=pl.DeviceIdType.MESH,
            )
            rdma.start()
            rdma.wait()

        compute_chunk(y, N_Y - 2, out_ref, True)

    return pl.pallas_call(
        body,
        out_shape=jax.ShapeDtypeStruct((chunk, f), jnp.float32),
        in_specs=[
            pl.BlockSpec(memory_space=pltpu.VMEM),
            pl.BlockSpec(memory_space=pltpu.VMEM),
        ],
        out_specs=pl.BlockSpec(memory_space=pltpu.VMEM),
        scratch_shapes=[
            pltpu.VMEM((chunk, f), jnp.bfloat16),
            pltpu.VMEM((N_Y - 1, chunk, f), jnp.bfloat16),
            pltpu.SemaphoreType.DMA,
            pltpu.SemaphoreType.DMA((N_Y - 1,)),
        ],
        compiler_params=pltpu.CompilerParams(collective_id=0),
    )(xt, dyb)


# baseline (device time: 455734 ns/iter reference)
import jax
import jax.numpy as jnp
from jax import lax
from jax.experimental import pallas as pl
from jax.experimental.pallas import tpu as pltpu

N_Y = 4
F_SLAB = 2048


def kernel(x, dy):
    m_shard, d = x.shape
    _, f = dy.shape
    chunk = d // N_Y
    n_slab = f // F_SLAB

    xt = x.T.astype(jnp.bfloat16)
    dyb = dy.astype(jnp.bfloat16)

    def body(xt_ref, dy_ref, out_ref, dybuf, stage, recv, ostage,
             dy_sems, out_sems, send_sem, recv_sems):
        y = lax.axis_index("y")
        xp = lax.axis_index("x")
        zp = lax.axis_index("z")
        left = (y + N_Y - 1) % N_Y
        right = (y + 1) % N_Y

        barrier = pltpu.get_barrier_semaphore()
        for nbr in (left, right):
            pl.semaphore_signal(
                barrier, inc=1,
                device_id=(xp, nbr, zp),
                device_id_type=pl.DeviceIdType.MESH,
            )
        pl.semaphore_wait(barrier, 2)

        def dy_load(s):
            return pltpu.make_async_copy(
                dy_ref.at[:, pl.ds(s * F_SLAB, F_SLAB)],
                dybuf.at[0],
                dy_sems.at[0],
            )

        for s in range(n_slab):
            q = s % 2
            dy_load(s).start()
            dy_load(s).wait()

            for h in range(N_Y - 1):
                c = (y + (N_Y - 1 - h)) % N_Y
                xrow = xt_ref[pl.ds(c * chunk, chunk), :]
                p = jnp.dot(xrow, dybuf[0], preferred_element_type=jnp.float32)
                if h > 0:
                    p = p + recv[q, h - 1].astype(jnp.float32)
                stage[:, :] = p.astype(jnp.bfloat16)
                rdma = pltpu.make_async_remote_copy(
                    src_ref=stage,
                    dst_ref=recv.at[q, h],
                    send_sem=send_sem,
                    recv_sem=recv_sems.at[q, h],
                    device_id=(xp, right, zp),
                    device_id_type=pl.DeviceIdType.MESH,
                )
                rdma.start()
                rdma.wait()

            xrow = xt_ref[pl.ds(y * chunk, chunk), :]
            ostage[0, :, :] = (
                jnp.dot(xrow, dybuf[0], preferred_element_type=jnp.float32)
                + recv[q, N_Y - 2].astype(jnp.float32)
            )
            ocopy = pltpu.make_async_copy(
                ostage.at[0],
                out_ref.at[:, pl.ds(s * F_SLAB, F_SLAB)],
                out_sems.at[0],
            )
            ocopy.start()
            ocopy.wait()

    return pl.pallas_call(
        body,
        out_shape=jax.ShapeDtypeStruct((chunk, f), jnp.float32),
        in_specs=[
            pl.BlockSpec(memory_space=pltpu.VMEM),
            pl.BlockSpec(memory_space=pl.ANY),
        ],
        out_specs=pl.BlockSpec(memory_space=pl.ANY),
        scratch_shapes=[
            pltpu.VMEM((1, m_shard, F_SLAB), jnp.bfloat16),
            pltpu.VMEM((chunk, F_SLAB), jnp.bfloat16),
            pltpu.VMEM((2, N_Y - 1, chunk, F_SLAB), jnp.bfloat16),
            pltpu.VMEM((1, chunk, F_SLAB), jnp.float32),
            pltpu.SemaphoreType.DMA((1,)),
            pltpu.SemaphoreType.DMA((1,)),
            pltpu.SemaphoreType.DMA,
            pltpu.SemaphoreType.DMA((2, N_Y - 1)),
        ],
        compiler_params=pltpu.CompilerParams(collective_id=0),
    )(xt, dyb)


# device time: 192377 ns/iter; 2.3690x vs baseline; 2.3690x over previous
import jax
import jax.numpy as jnp
from jax import lax
from jax.experimental import pallas as pl
from jax.experimental.pallas import tpu as pltpu

N_Y = 4
N_SLAB = 2
F_SLAB = 1024


def kernel(x, dy):
    m_shard, d = x.shape
    _, f = dy.shape
    chunk = d // N_Y
    part_w = f // 4

    xt = x.T.astype(jnp.bfloat16)
    dyb = dy.astype(jnp.bfloat16)

    def body(xt_ref, dy_ref, out_ref, dybuf, stage, yrecv, xsend, xrecv,
             ostage, dy_sem, ysend_sems, yrecv_sems, xsend_sems, xrecv_sems,
             out_sem):
        y = lax.axis_index("y")
        xp = lax.axis_index("x")
        zp = lax.axis_index("z")
        left = (y + N_Y - 1) % N_Y
        right = (y + 1) % N_Y
        zq = zp % 2
        zn = zp + 1 - 2 * zq
        part = 2 * zq + xp
        base = part * part_w

        partners = [
            ((1 - xp, y, zp), 2 * zq + (1 - xp)),
            ((xp, y, zn), 2 * (1 - zq) + xp),
            ((1 - xp, y, zn), 2 * (1 - zq) + (1 - xp)),
        ]

        barrier = pltpu.get_barrier_semaphore()
        for dev in [(xp, left, zp), (xp, right, zp)] + [p[0] for p in partners]:
            pl.semaphore_signal(barrier, inc=1, device_id=dev,
                                device_id_type=pl.DeviceIdType.MESH)
        pl.semaphore_wait(barrier, 5)

        dy_load = pltpu.make_async_copy(
            dy_ref.at[:, pl.ds(base, part_w)], dybuf, dy_sem)
        dy_load.start()
        dy_load.wait()

        def make_yrdma(g):
            s, h = divmod(g, N_Y - 1)
            return pltpu.make_async_remote_copy(
                src_ref=stage.at[g % 2],
                dst_ref=yrecv.at[s, h],
                send_sem=ysend_sems.at[g % 2],
                recv_sem=yrecv_sems.at[s, h],
                device_id=(xp, right, zp),
                device_id_type=pl.DeviceIdType.MESH,
            )

        def make_xrdma(s, j):
            return pltpu.make_async_remote_copy(
                src_ref=xsend,
                dst_ref=xrecv.at[s, j],
                send_sem=xsend_sems.at[s, j],
                recv_sem=xrecv_sems.at[s, j],
                device_id=partners[j][0],
                device_id_type=pl.DeviceIdType.MESH,
            )

        def dot_chunk(c, sl):
            xrow = xt_ref[pl.ds(c * chunk, chunk), :]
            return jnp.dot(xrow, dybuf[:, sl],
                           preferred_element_type=jnp.float32)

        for s in range(N_SLAB):
            sl = pl.ds(s * F_SLAB, F_SLAB)
            for h in range(N_Y - 1):
                g = s * (N_Y - 1) + h
                c = (y + (N_Y - 1 - h)) % N_Y
                p = dot_chunk(c, sl)
                if h > 0:
                    make_yrdma(g - 1).wait_recv()
                    p = p + yrecv[s, h - 1, :, :].astype(jnp.float32)
                if g >= 2:
                    make_yrdma(g - 2).wait_send()
                stage[g % 2, :, :] = p.astype(jnp.bfloat16)
                make_yrdma(g).start()

            make_yrdma(s * (N_Y - 1) + 2).wait_recv()
            fin = (dot_chunk(y, sl)
                   + yrecv[s, N_Y - 2, :, :].astype(jnp.float32))

            if s > 0:
                pltpu.make_async_copy(
                    ostage, out_ref.at[:, pl.ds(0, F_SLAB)], out_sem
                ).wait()
            ostage[:, :] = fin
            pltpu.make_async_copy(
                ostage,
                out_ref.at[:, pl.ds(base + s * F_SLAB, F_SLAB)],
                out_sem,
            ).start()

            if s > 0:
                for j in range(3):
                    make_xrdma(s - 1, j).wait_send()
            xsend[:, :] = fin.astype(jnp.bfloat16)
            for j in range(3):
                make_xrdma(s, j).start()

        for s in range(N_SLAB):
            for j in range(3):
                make_xrdma(s, j).wait_recv()
                pltpu.make_async_copy(
                    ostage, out_ref.at[:, pl.ds(0, F_SLAB)], out_sem
                ).wait()
                ostage[:, :] = xrecv[s, j, :, :].astype(jnp.float32)
                col = partners[j][1] * part_w + s * F_SLAB
                pltpu.make_async_copy(
                    ostage,
                    out_ref.at[:, pl.ds(col, F_SLAB)],
                    out_sem,
                ).start()

        total_g = N_SLAB * (N_Y - 1)
        for g in (total_g - 2, total_g - 1):
            make_yrdma(g).wait_send()
        for j in range(3):
            make_xrdma(N_SLAB - 1, j).wait_send()
        pltpu.make_async_copy(
            ostage, out_ref.at[:, pl.ds(0, F_SLAB)], out_sem
        ).wait()

    return pl.pallas_call(
        body,
        out_shape=jax.ShapeDtypeStruct((chunk, f), jnp.float32),
        in_specs=[
            pl.BlockSpec(memory_space=pltpu.VMEM),
            pl.BlockSpec(memory_space=pl.ANY),
        ],
        out_specs=pl.BlockSpec(memory_space=pl.ANY),
        scratch_shapes=[
            pltpu.VMEM((m_shard, part_w), jnp.bfloat16),
            pltpu.VMEM((2, chunk, F_SLAB), jnp.bfloat16),
            pltpu.VMEM((N_SLAB, N_Y - 1, chunk, F_SLAB), jnp.bfloat16),
            pltpu.VMEM((chunk, F_SLAB), jnp.bfloat16),
            pltpu.VMEM((N_SLAB, 3, chunk, F_SLAB), jnp.bfloat16),
            pltpu.VMEM((chunk, F_SLAB), jnp.float32),
            pltpu.SemaphoreType.DMA,
            pltpu.SemaphoreType.DMA((2,)),
            pltpu.SemaphoreType.DMA((N_SLAB, N_Y - 1)),
            pltpu.SemaphoreType.DMA((N_SLAB, 3)),
            pltpu.SemaphoreType.DMA((N_SLAB, 3)),
            pltpu.SemaphoreType.DMA,
        ],
        compiler_params=pltpu.CompilerParams(collective_id=0),
    )(xt, dyb)


# device time: 158118 ns/iter; 2.8822x vs baseline; 1.2167x over previous
import jax
import jax.numpy as jnp
from jax import lax
from jax.experimental import pallas as pl
from jax.experimental.pallas import tpu as pltpu

N_Y = 4
N_SLAB = 2
F_SLAB = 1024


def kernel(x, dy):
    m_shard, d = x.shape
    _, f = dy.shape
    chunk = d // N_Y
    part_w = f // 4


    def body(xb_ref, dy_ref, out_ref, dyf32, stage, yrecv, xsend, xrecv,
             ostage, dy_sem, ysend_sems, yrecv_sems, xsend_sems, xrecv_sems,
             out_sem):
        y = lax.axis_index("y")
        xp = lax.axis_index("x")
        zp = lax.axis_index("z")
        left = (y + N_Y - 1) % N_Y
        right = (y + 1) % N_Y
        zq = zp % 2
        zn = zp + 1 - 2 * zq
        part = 2 * zq + xp
        base = part * part_w

        partners = [
            ((1 - xp, y, zp), 2 * zq + (1 - xp)),
            ((xp, y, zn), 2 * (1 - zq) + xp),
            ((1 - xp, y, zn), 2 * (1 - zq) + (1 - xp)),
        ]

        barrier = pltpu.get_barrier_semaphore()
        for dev in [(xp, left, zp), (xp, right, zp)] + [p[0] for p in partners]:
            pl.semaphore_signal(barrier, inc=1, device_id=dev,
                                device_id_type=pl.DeviceIdType.MESH)
        pl.semaphore_wait(barrier, 5)

        def dy_load(s):
            return pltpu.make_async_copy(
                dy_ref.at[:, pl.ds(base + s * F_SLAB, F_SLAB)], dyf32, dy_sem)

        dy_load(0).start()

        def make_yrdma(g):
            s, h = divmod(g, N_Y - 1)
            return pltpu.make_async_remote_copy(
                src_ref=stage.at[g % 2],
                dst_ref=yrecv.at[s, h],
                send_sem=ysend_sems.at[g % 2],
                recv_sem=yrecv_sems.at[s, h],
                device_id=(xp, right, zp),
                device_id_type=pl.DeviceIdType.MESH,
            )

        def make_xrdma(s, j):
            return pltpu.make_async_remote_copy(
                src_ref=xsend,
                dst_ref=xrecv.at[s, j],
                send_sem=xsend_sems.at[s, j],
                recv_sem=xrecv_sems.at[s, j],
                device_id=partners[j][0],
                device_id_type=pl.DeviceIdType.MESH,
            )

        def dot_chunk(c):
            xcols = xb_ref[:, pl.ds(c * chunk, chunk)]
            return lax.dot_general(
                xcols, dyf32[:, :],
                (((0,), (0,)), ((), ())),
                preferred_element_type=jnp.float32,
            )

        for s in range(N_SLAB):
            dy_load(s).wait()
            for h in range(N_Y - 1):
                g = s * (N_Y - 1) + h
                c = (y + (N_Y - 1 - h)) % N_Y
                p = dot_chunk(c)
                if h > 0:
                    make_yrdma(g - 1).wait_recv()
                    p = p + yrecv[s, h - 1, :, :].astype(jnp.float32)
                if g >= 2:
                    make_yrdma(g - 2).wait_send()
                stage[g % 2, :, :] = p.astype(jnp.bfloat16)
                make_yrdma(g).start()

            make_yrdma(s * (N_Y - 1) + 2).wait_recv()
            fin = (dot_chunk(y)
                   + yrecv[s, N_Y - 2, :, :].astype(jnp.float32))
            if s + 1 < N_SLAB:
                dy_load(s + 1).start()

            if s > 0:
                pltpu.make_async_copy(
                    ostage, out_ref.at[:, pl.ds(0, F_SLAB)], out_sem
                ).wait()
            ostage[:, :] = fin
            pltpu.make_async_copy(
                ostage,
                out_ref.at[:, pl.ds(base + s * F_SLAB, F_SLAB)],
                out_sem,
            ).start()

            if s > 0:
                for j in range(3):
                    make_xrdma(s - 1, j).wait_send()
            xsend[:, :] = fin.astype(jnp.bfloat16)
            for j in range(3):
                make_xrdma(s, j).start()

        for s in range(N_SLAB):
            for j in range(3):
                make_xrdma(s, j).wait_recv()
                pltpu.make_async_copy(
                    ostage, out_ref.at[:, pl.ds(0, F_SLAB)], out_sem
                ).wait()
                ostage[:, :] = xrecv[s, j, :, :].astype(jnp.float32)
                col = partners[j][1] * part_w + s * F_SLAB
                pltpu.make_async_copy(
                    ostage,
                    out_ref.at[:, pl.ds(col, F_SLAB)],
                    out_sem,
                ).start()

        total_g = N_SLAB * (N_Y - 1)
        for g in (total_g - 2, total_g - 1):
            make_yrdma(g).wait_send()
        for j in range(3):
            make_xrdma(N_SLAB - 1, j).wait_send()
        pltpu.make_async_copy(
            ostage, out_ref.at[:, pl.ds(0, F_SLAB)], out_sem
        ).wait()

    return pl.pallas_call(
        body,
        out_shape=jax.ShapeDtypeStruct((chunk, f), jnp.float32),
        in_specs=[
            pl.BlockSpec(memory_space=pltpu.VMEM),
            pl.BlockSpec(memory_space=pl.ANY),
        ],
        out_specs=pl.BlockSpec(memory_space=pl.ANY),
        scratch_shapes=[
            pltpu.VMEM((m_shard, F_SLAB), jnp.float32),
            pltpu.VMEM((2, chunk, F_SLAB), jnp.bfloat16),
            pltpu.VMEM((N_SLAB, N_Y - 1, chunk, F_SLAB), jnp.bfloat16),
            pltpu.VMEM((chunk, F_SLAB), jnp.bfloat16),
            pltpu.VMEM((N_SLAB, 3, chunk, F_SLAB), jnp.bfloat16),
            pltpu.VMEM((chunk, F_SLAB), jnp.float32),
            pltpu.SemaphoreType.DMA,
            pltpu.SemaphoreType.DMA((2,)),
            pltpu.SemaphoreType.DMA((N_SLAB, N_Y - 1)),
            pltpu.SemaphoreType.DMA((N_SLAB, 3)),
            pltpu.SemaphoreType.DMA((N_SLAB, 3)),
            pltpu.SemaphoreType.DMA,
        ],
        compiler_params=pltpu.CompilerParams(collective_id=0),
    )(x, dy)


# device time: 140555 ns/iter; 3.2424x vs baseline; 1.1250x over previous
import jax
import jax.numpy as jnp
from jax import lax
from jax.experimental import pallas as pl
from jax.experimental.pallas import tpu as pltpu

N_Y = 4
N_SLAB = 2
F_SLAB = 1024


def kernel(x, dy):
    m_shard, d = x.shape
    _, f = dy.shape
    chunk = d // N_Y
    part_w = f // 4


    def body(xb_ref, dy_ref, out_ref, dyf32, stage, yrecv, xsend, xrecv,
             ostage, dy_sems, ysend_sems, yrecv_sems, xsend_sems, xrecv_sems,
             out_sems):
        y = lax.axis_index("y")
        xp = lax.axis_index("x")
        zp = lax.axis_index("z")
        left = (y + N_Y - 1) % N_Y
        right = (y + 1) % N_Y
        zq = zp % 2
        zn = zp + 1 - 2 * zq
        part = 2 * zq + xp
        base = part * part_w

        partners = [
            ((1 - xp, y, zp), 2 * zq + (1 - xp)),
            ((xp, y, zn), 2 * (1 - zq) + xp),
            ((1 - xp, y, zn), 2 * (1 - zq) + (1 - xp)),
        ]

        barrier = pltpu.get_barrier_semaphore()
        for dev in [(xp, left, zp), (xp, right, zp)] + [p[0] for p in partners]:
            pl.semaphore_signal(barrier, inc=1, device_id=dev,
                                device_id_type=pl.DeviceIdType.MESH)
        pl.semaphore_wait(barrier, 5)

        def dy_load(s):
            return pltpu.make_async_copy(
                dy_ref.at[:, pl.ds(base + s * F_SLAB, F_SLAB)],
                dyf32.at[s], dy_sems.at[s])

        dy_load(0).start()
        dy_load(1).start()

        WAVE = [(s, h) for h in range(N_Y - 1) for s in range(N_SLAB)]

        def make_yrdma(g):
            s, h = WAVE[g]
            return pltpu.make_async_remote_copy(
                src_ref=stage.at[g % 2],
                dst_ref=yrecv.at[s, h],
                send_sem=ysend_sems.at[g % 2],
                recv_sem=yrecv_sems.at[s, h],
                device_id=(xp, right, zp),
                device_id_type=pl.DeviceIdType.MESH,
            )

        def make_yrecv_wait(s, h):
            return pltpu.make_async_remote_copy(
                src_ref=stage.at[0],
                dst_ref=yrecv.at[s, h],
                send_sem=ysend_sems.at[0],
                recv_sem=yrecv_sems.at[s, h],
                device_id=(xp, right, zp),
                device_id_type=pl.DeviceIdType.MESH,
            )

        def make_xrdma(s, j):
            return pltpu.make_async_remote_copy(
                src_ref=xsend,
                dst_ref=xrecv.at[s, j],
                send_sem=xsend_sems.at[s, j],
                recv_sem=xrecv_sems.at[s, j],
                device_id=partners[j][0],
                device_id_type=pl.DeviceIdType.MESH,
            )

        def dot_chunk(c, s):
            xcols = xb_ref[:, pl.ds(c * chunk, chunk)]
            return lax.dot_general(
                xcols, dyf32[s, :, :],
                (((0,), (0,)), ((), ())),
                preferred_element_type=jnp.float32,
            )

        dy_ready = set()
        for g, (s, h) in enumerate(WAVE):
            if s not in dy_ready:
                dy_load(s).wait()
                dy_ready.add(s)
            c = (y + (N_Y - 1 - h)) % N_Y
            p = dot_chunk(c, s)
            if h > 0:
                make_yrecv_wait(s, h - 1).wait_recv()
                p = p + yrecv[s, h - 1, :, :].astype(jnp.float32)
            if g >= 2:
                make_yrdma(g - 2).wait_send()
            stage[g % 2, :, :] = p.astype(jnp.bfloat16)
            make_yrdma(g).start()

        for s in range(N_SLAB):
            make_yrecv_wait(s, N_Y - 2).wait_recv()
            fin = (dot_chunk(y, s)
                   + yrecv[s, N_Y - 2, :, :].astype(jnp.float32))

            ostage[s, :, :] = fin
            pltpu.make_async_copy(
                ostage.at[s],
                out_ref.at[:, pl.ds(base + s * F_SLAB, F_SLAB)],
                out_sems.at[s],
            ).start()

            if s > 0:
                for j in range(3):
                    make_xrdma(s - 1, j).wait_send()
            xsend[:, :] = fin.astype(jnp.bfloat16)
            for j in range(3):
                make_xrdma(s, j).start()

        for s in range(N_SLAB):
            for j in range(3):
                make_xrdma(s, j).wait_recv()
                q = (s * 3 + j) % 2
                pltpu.make_async_copy(
                    ostage.at[q], out_ref.at[:, pl.ds(0, F_SLAB)],
                    out_sems.at[q],
                ).wait()
                ostage[q, :, :] = xrecv[s, j, :, :].astype(jnp.float32)
                col = partners[j][1] * part_w + s * F_SLAB
                pltpu.make_async_copy(
                    ostage.at[q],
                    out_ref.at[:, pl.ds(col, F_SLAB)],
                    out_sems.at[q],
                ).start()

        total_g = len(WAVE)
        for g in (total_g - 2, total_g - 1):
            make_yrdma(g).wait_send()
        for j in range(3):
            make_xrdma(N_SLAB - 1, j).wait_send()
        for q in range(2):
            pltpu.make_async_copy(
                ostage.at[q], out_ref.at[:, pl.ds(0, F_SLAB)], out_sems.at[q]
            ).wait()

    return pl.pallas_call(
        body,
        out_shape=jax.ShapeDtypeStruct((chunk, f), jnp.float32),
        in_specs=[
            pl.BlockSpec(memory_space=pltpu.VMEM),
            pl.BlockSpec(memory_space=pl.ANY),
        ],
        out_specs=pl.BlockSpec(memory_space=pl.ANY),
        scratch_shapes=[
            pltpu.VMEM((N_SLAB, m_shard, F_SLAB), jnp.float32),
            pltpu.VMEM((2, chunk, F_SLAB), jnp.bfloat16),
            pltpu.VMEM((N_SLAB, N_Y - 1, chunk, F_SLAB), jnp.bfloat16),
            pltpu.VMEM((chunk, F_SLAB), jnp.bfloat16),
            pltpu.VMEM((N_SLAB, 3, chunk, F_SLAB), jnp.bfloat16),
            pltpu.VMEM((2, chunk, F_SLAB), jnp.float32),
            pltpu.SemaphoreType.DMA((N_SLAB,)),
            pltpu.SemaphoreType.DMA((2,)),
            pltpu.SemaphoreType.DMA((N_SLAB, N_Y - 1)),
            pltpu.SemaphoreType.DMA((N_SLAB, 3)),
            pltpu.SemaphoreType.DMA((N_SLAB, 3)),
            pltpu.SemaphoreType.DMA((2,)),
        ],
        compiler_params=pltpu.CompilerParams(
            collective_id=0,
            vmem_limit_bytes=66_846_720,
        ),
    )(x, dy)
